# Initial kernel scaffold; baseline (speedup 1.0000x reference)
import jax
import jax.numpy as jnp
from jax.experimental import pallas as pl


def kernel(x, gate_w, lattice_weights, out_w, out_b, ln_gamma, ln_beta):
    raise NotImplementedError("write your pallas kernel here")


# speedup vs baseline: 2.3103x; 2.3103x over previous
import functools

import jax
import jax.numpy as jnp
from jax.experimental import pallas as pl
from jax.experimental.pallas import tpu as pltpu

S = 256
D = 768
L = 48
K = 4


def _fused_kernel(x_ref, gate_w_ref, w_ref, out_w_ref, out_b_ref,
                  ln_g_ref, ln_b_ref, o_ref, g_ref, acc_ref):
    l = pl.program_id(0)

    @pl.when(l == 0)
    def _gate():
        x = x_ref[...]
        logits = jax.lax.dot_general(
            x, gate_w_ref[...], (((1,), (1,)), ((), ())),
            preferred_element_type=jnp.float32)
        lane = jax.lax.broadcasted_iota(jnp.int32, (S, L), 1)
        work = logits
        neg_inf = jnp.float32(-jnp.inf)
        vals = []
        sels = []
        for _ in range(K):
            m = jnp.max(work, axis=-1, keepdims=True)
            is_m = work >= m
            first = jnp.min(jnp.where(is_m, lane, L), axis=-1,
                            keepdims=True)
            sel = lane == first
            vals.append(m)
            sels.append(sel)
            work = jnp.where(sel, neg_inf, work)
        v = jnp.concatenate(vals, axis=-1)
        mx = jnp.max(v, axis=-1, keepdims=True)
        e = jnp.exp(v - mx)
        p = e / jnp.sum(e, axis=-1, keepdims=True)
        g = jnp.zeros((S, L), jnp.float32)
        for j in range(K):
            g = g + jnp.where(sels[j], p[:, j:j + 1], 0.0)
        g_ref[...] = g
        acc_ref[...] = jnp.zeros((S, D), jnp.float32)

    lane = jax.lax.broadcasted_iota(jnp.int32, (S, L), 1)
    g_col = jnp.sum(jnp.where(lane == l, g_ref[...], 0.0), axis=-1,
                    keepdims=True)
    y = jax.lax.dot_general(
        x_ref[...], w_ref[0], (((1,), (0,)), ((), ())),
        preferred_element_type=jnp.float32)
    acc_ref[...] += g_col * y

    @pl.when(l == L - 1)
    def _epilogue():
        x = x_ref[...]
        h = x + jax.lax.dot_general(
            acc_ref[...], out_w_ref[...], (((1,), (1,)), ((), ())),
            preferred_element_type=jnp.float32) + out_b_ref[...]
        mean = jnp.mean(h, axis=-1, keepdims=True)
        c = h - mean
        var = jnp.mean(c * c, axis=-1, keepdims=True)
        o_ref[...] = c * jax.lax.rsqrt(var + 1e-5) * ln_g_ref[...] + ln_b_ref[...]


@functools.partial(jax.jit, static_argnames=())
def kernel(x, gate_w, lattice_weights, out_w, out_b, ln_gamma, ln_beta):
    x2 = x.reshape(S, D)
    out = pl.pallas_call(
        _fused_kernel,
        grid=(L,),
        in_specs=[
            pl.BlockSpec((S, D), lambda l: (0, 0)),
            pl.BlockSpec((L, D), lambda l: (0, 0)),
            pl.BlockSpec((1, D, D), lambda l: (l, 0, 0)),
            pl.BlockSpec((D, D), lambda l: (0, 0)),
            pl.BlockSpec((1, D), lambda l: (0, 0)),
            pl.BlockSpec((1, D), lambda l: (0, 0)),
            pl.BlockSpec((1, D), lambda l: (0, 0)),
        ],
        out_specs=pl.BlockSpec((S, D), lambda l: (0, 0)),
        out_shape=jax.ShapeDtypeStruct((S, D), jnp.float32),
        scratch_shapes=[
            pltpu.VMEM((S, L), jnp.float32),
            pltpu.VMEM((S, D), jnp.float32),
        ],
        compiler_params=pltpu.CompilerParams(
            dimension_semantics=("arbitrary",),
        ),
    )(x2, gate_w, lattice_weights, out_w, out_b.reshape(1, D),
      ln_gamma.reshape(1, D), ln_beta.reshape(1, D))
    return out.reshape(1, S, D)
